# baseline (speedup 1.0000x reference)
import jax
import jax.numpy as jnp
from jax import lax
from jax.experimental import pallas as pl
from jax.experimental.pallas import tpu as pltpu

B = 16384
C = 1000

LAMB = max(5.0, 1500.0 / (1.0 + 0.1 * 1))
INV = 1.0 / (1.0 + LAMB)

_R = 1024
_NT = B // _R


def _body(cos_ref, phi_ref, tgt_ref, out_ref):
    x = cos_ref[...]
    p = phi_ref[...]
    t = tgt_ref[...]
    col = lax.broadcasted_iota(jnp.int32, x.shape, 1)
    onehot = col == t
    out = jnp.where(onehot, x - x * INV + p * INV, x)
    m = jnp.max(out, axis=1, keepdims=True)
    s = jnp.sum(jnp.exp(out - m), axis=1, keepdims=True)
    out_t = jnp.sum(jnp.where(onehot, out, 0.0), axis=1, keepdims=True)
    out_ref[...] = jnp.sum(m + jnp.log(s) - out_t).reshape(1, 1, 1)


_call = pl.pallas_call(
    _body,
    grid=(_NT,),
    in_specs=[
        pl.BlockSpec((_R, C), lambda i: (i, 0)),
        pl.BlockSpec((_R, C), lambda i: (i, 0)),
        pl.BlockSpec((_R, 1), lambda i: (i, 0)),
    ],
    out_specs=pl.BlockSpec((1, 1, 1), lambda i: (i, 0, 0)),
    out_shape=jax.ShapeDtypeStruct((_NT, 1, 1), jnp.float32),
    compiler_params=pltpu.CompilerParams(
        dimension_semantics=("parallel",),
    ),
)


def _mean_body(part_ref, out_ref):
    out_ref[...] = (jnp.sum(part_ref[...]) * (1.0 / B)).reshape(1, 1)


_mean_call = pl.pallas_call(
    _mean_body,
    out_shape=jax.ShapeDtypeStruct((1, 1), jnp.float32),
)


def kernel(cos_theta, phi_theta, target):
    tgt = target.reshape(-1).astype(jnp.int32).reshape(B, 1)
    part = _call(cos_theta, phi_theta, tgt)
    return _mean_call(part).reshape(())


# speedup vs baseline: 1.0224x; 1.0224x over previous
import jax
import jax.numpy as jnp
from jax import lax
from jax.experimental import pallas as pl

B = 16384
C = 1000

LAMB = max(5.0, 1500.0 / (1.0 + 0.1 * 1))
INV = 1.0 / (1.0 + LAMB)

_R = 2048
_NT = B // _R


def _body(cos_ref, phi_ref, tgt_ref, out_ref):
    x = cos_ref[...]
    p = phi_ref[...]
    t = tgt_ref[...]
    col = lax.broadcasted_iota(jnp.int32, x.shape, 1)
    onehot = col == t
    out = jnp.where(onehot, x - x * INV + p * INV, x)
    m = jnp.max(out, axis=1, keepdims=True)
    s = jnp.sum(jnp.exp(out - m), axis=1, keepdims=True)
    out_t = jnp.sum(jnp.where(onehot, out, 0.0), axis=1, keepdims=True)
    tile_loss = jnp.sum(m + jnp.log(s) - out_t, axis=0, keepdims=True) * (1.0 / B)

    @pl.when(pl.program_id(0) == 0)
    def _():
        out_ref[...] = jnp.zeros_like(out_ref)

    out_ref[...] += tile_loss


_call = pl.pallas_call(
    _body,
    grid=(_NT,),
    in_specs=[
        pl.BlockSpec((_R, C), lambda i: (i, 0)),
        pl.BlockSpec((_R, C), lambda i: (i, 0)),
        pl.BlockSpec((_R, 1), lambda i: (i, 0)),
    ],
    out_specs=pl.BlockSpec((1, 1), lambda i: (0, 0)),
    out_shape=jax.ShapeDtypeStruct((1, 1), jnp.float32),
)


def kernel(cos_theta, phi_theta, target):
    tgt = target.reshape(-1).astype(jnp.int32)
    out = _call(cos_theta, phi_theta, tgt.reshape(B, 1))
    return out.reshape(())
